# Initial kernel scaffold; baseline (speedup 1.0000x reference)
import jax
import jax.numpy as jnp
from jax.experimental import pallas as pl


def kernel(edge_feats, node_feats, edge_index, edge_W1, edge_b1, edge_W2, edge_b2, edge_ln_g, edge_ln_b, node_W1, node_b1, node_W2, node_b2, node_ln_g, node_ln_b):
    raise NotImplementedError("write your pallas kernel here")


# speedup vs baseline: 2.1220x; 2.1220x over previous
import functools

import jax
import jax.numpy as jnp
from jax import lax
from jax.experimental import pallas as pl
from jax.experimental.pallas import tpu as pltpu
from jax.experimental.pallas import tpu_sc as plsc

_NC = 2
_NS = 16
_NW = _NC * _NS
_LANE = 16

_LN_EPS = 1e-5


def _sc_mesh():
  return plsc.VectorSubcoreMesh(
      core_axis_name="c", subcore_axis_name="s",
      num_cores=_NC, num_subcores=_NS)


def _make_gather_add(E, D, CH):
  ew = E // _NW
  nchunk = ew // CH

  @functools.partial(
      pl.kernel,
      out_type=jax.ShapeDtypeStruct((E, D), jnp.float32),
      mesh=_sc_mesh(),
      scratch_types=[
          pltpu.VMEM((CH,), jnp.int32),
          pltpu.VMEM((CH,), jnp.int32),
          pltpu.VMEM((CH, D), jnp.float32),
          pltpu.VMEM((CH, D), jnp.float32),
          pltpu.SemaphoreType.DMA,
          pltpu.SemaphoreType.DMA,
      ],
  )
  def gather_kernel(p_hbm, q_hbm, src_hbm, dst_hbm, out_hbm,
                    idx_s, idx_d, rows_p, rows_q, sem_p, sem_q):
    c = lax.axis_index("c")
    s = lax.axis_index("s")
    wid = s * _NC + c
    base = wid * ew

    def chunk_body(j, carry):
      off = pl.multiple_of(base + j * CH, 8)
      pltpu.sync_copy(src_hbm.at[pl.ds(off, CH)], idx_s)
      pltpu.sync_copy(dst_hbm.at[pl.ds(off, CH)], idx_d)
      cp_p = pltpu.async_copy(p_hbm.at[idx_s], rows_p, sem_p)
      cp_q = pltpu.async_copy(q_hbm.at[idx_d], rows_q, sem_q)
      cp_p.wait()
      cp_q.wait()

      def add_row(r, carry2):
        for k in range(D // _LANE):
          sl = pl.ds(k * _LANE, _LANE)
          rows_p[r, sl] = rows_p[r, sl] + rows_q[r, sl]
        return carry2

      lax.fori_loop(0, CH, add_row, 0, unroll=2)
      pltpu.sync_copy(rows_p, out_hbm.at[pl.ds(off, CH)])
      return carry

    lax.fori_loop(0, nchunk, chunk_body, 0)

  return gather_kernel


def _make_scatter_agg(E, N, D, CH):
  ew = E // _NW
  nchunk = ew // CH
  zrows = 200
  wsub = 10
  rows_per_sub = N // wsub
  nzcopy = rows_per_sub // zrows
  assert wsub * rows_per_sub == N and nzcopy * zrows == rows_per_sub

  @functools.partial(
      pl.kernel,
      out_type=jax.ShapeDtypeStruct((_NC, N, D), jnp.float32),
      mesh=_sc_mesh(),
      scratch_types=[
          pltpu.VMEM((1, CH), jnp.int32),
          pltpu.VMEM((CH, D), jnp.float32),
          pltpu.VMEM((zrows, D), jnp.float32),
          pltpu.VMEM_SHARED((N, D), jnp.float32),
      ],
  )
  def scatter_kernel(edge_hbm, dst_hbm, out_hbm, idx_v, rows_v, zbuf, agg_sh):
    c = lax.axis_index("c")
    s = lax.axis_index("s")
    wid = s * _NC + c
    base = wid * ew

    @pl.when(s < wsub)
    def _zero():
      def zero_row(i, carry):
        for k in range(D // _LANE):
          zbuf[i, pl.ds(k * _LANE, _LANE)] = jnp.zeros((_LANE,), jnp.float32)
        return carry

      lax.fori_loop(0, zrows, zero_row, 0)

      def zero_copy(t, carry):
        r0 = pl.multiple_of(s * rows_per_sub + t * zrows, 8)
        pltpu.sync_copy(zbuf, agg_sh.at[pl.ds(r0, zrows)])
        return carry

      lax.fori_loop(0, nzcopy, zero_copy, 0)

    plsc.subcore_barrier()

    def chunk_body(j, carry):
      off = pl.multiple_of(base + j * CH, 8)
      pltpu.sync_copy(dst_hbm.at[pl.ds(off, CH)], idx_v.at[0])
      pltpu.sync_copy(edge_hbm.at[pl.ds(off, CH)], rows_v)
      pltpu.sync_copy(rows_v, agg_sh.at[idx_v.at[0]], add=True)
      return carry

    lax.fori_loop(0, nchunk, chunk_body, 0)
    plsc.subcore_barrier()

    @pl.when(s < wsub)
    def _writeout():
      r0 = pl.multiple_of(s * rows_per_sub, 8)
      pltpu.sync_copy(agg_sh.at[pl.ds(r0, rows_per_sub)],
                      out_hbm.at[c, pl.ds(r0, rows_per_sub)])

  return scatter_kernel


def _pq_body(x_ref, ws_ref, wd_ref, p_ref, q_ref):
  x = x_ref[...]
  p_ref[...] = jnp.dot(x, ws_ref[...], preferred_element_type=jnp.float32)
  q_ref[...] = jnp.dot(x, wd_ref[...], preferred_element_type=jnp.float32)


def _layer_norm(h, g, b):
  m = jnp.mean(h, axis=-1, keepdims=True)
  xc = h - m
  v = jnp.mean(xc * xc, axis=-1, keepdims=True)
  return xc * lax.rsqrt(v + _LN_EPS) * g + b


def _edge_mlp_body(e_ref, g_ref, w1_ref, w2_ref, b1_ref, b2_ref,
                   lng_ref, lnb_ref, out_ref):
  x = e_ref[...]
  cvt = jnp.dot(x, w1_ref[...], preferred_element_type=jnp.float32)
  cvt = cvt + g_ref[...] + b1_ref[...]
  h = cvt * jax.nn.sigmoid(cvt)
  y = jnp.dot(h, w2_ref[...], preferred_element_type=jnp.float32) + b2_ref[...]
  out_ref[...] = x + _layer_norm(y, lng_ref[...], lnb_ref[...])


def _node_mlp_body(x_ref, a0_ref, a1_ref, w1x_ref, w1a_ref, w2_ref,
                   b1_ref, b2_ref, lng_ref, lnb_ref, out_ref):
  x = x_ref[...]
  a = a0_ref[...] + a1_ref[...]
  cvt = jnp.dot(x, w1x_ref[...], preferred_element_type=jnp.float32)
  cvt = cvt + jnp.dot(a, w1a_ref[...], preferred_element_type=jnp.float32)
  cvt = cvt + b1_ref[...]
  h = cvt * jax.nn.sigmoid(cvt)
  y = jnp.dot(h, w2_ref[...], preferred_element_type=jnp.float32) + b2_ref[...]
  out_ref[...] = x + _layer_norm(y, lng_ref[...], lnb_ref[...])


def _mat_spec(D):
  return pl.BlockSpec((D, D), lambda i: (0, 0))


def _vec_spec(D):
  return pl.BlockSpec((1, D), lambda i: (0, 0))


def kernel(edge_feats, node_feats, edge_index,
           edge_W1, edge_b1, edge_W2, edge_b2, edge_ln_g, edge_ln_b,
           node_W1, node_b1, node_W2, node_b2, node_ln_g, node_ln_b):
  E, D = edge_feats.shape
  N = node_feats.shape[0]
  L = edge_W1.shape[0]
  src = edge_index[0]
  dst = edge_index[1]

  CH = 80
  EBLK = 2000
  NBLK = 1000

  gather_add = _make_gather_add(E, D, CH)
  scatter_agg = _make_scatter_agg(E, N, D, CH)

  pq_call = pl.pallas_call(
      _pq_body,
      grid=(N // NBLK,),
      in_specs=[pl.BlockSpec((NBLK, D), lambda i: (i, 0)),
                _mat_spec(D), _mat_spec(D)],
      out_specs=[pl.BlockSpec((NBLK, D), lambda i: (i, 0)),
                 pl.BlockSpec((NBLK, D), lambda i: (i, 0))],
      out_shape=[jax.ShapeDtypeStruct((N, D), jnp.float32),
                 jax.ShapeDtypeStruct((N, D), jnp.float32)],
  )

  edge_mlp = pl.pallas_call(
      _edge_mlp_body,
      grid=(E // EBLK,),
      in_specs=[pl.BlockSpec((EBLK, D), lambda i: (i, 0)),
                pl.BlockSpec((EBLK, D), lambda i: (i, 0)),
                _mat_spec(D), _mat_spec(D),
                _vec_spec(D), _vec_spec(D), _vec_spec(D), _vec_spec(D)],
      out_specs=pl.BlockSpec((EBLK, D), lambda i: (i, 0)),
      out_shape=jax.ShapeDtypeStruct((E, D), jnp.float32),
  )

  node_mlp = pl.pallas_call(
      _node_mlp_body,
      grid=(N // NBLK,),
      in_specs=[pl.BlockSpec((NBLK, D), lambda i: (i, 0)),
                pl.BlockSpec((NBLK, D), lambda i: (i, 0)),
                pl.BlockSpec((NBLK, D), lambda i: (i, 0)),
                _mat_spec(D), _mat_spec(D), _mat_spec(D),
                _vec_spec(D), _vec_spec(D), _vec_spec(D), _vec_spec(D)],
      out_specs=pl.BlockSpec((NBLK, D), lambda i: (i, 0)),
      out_shape=jax.ShapeDtypeStruct((N, D), jnp.float32),
  )

  for i in range(L):
    w1e = edge_W1[i, :D]
    w1s = edge_W1[i, D:2 * D]
    w1d = edge_W1[i, 2 * D:]
    p, q = pq_call(node_feats, w1s, w1d)
    g = gather_add(p, q, src, dst)
    edge_feats = edge_mlp(
        edge_feats, g, w1e, edge_W2[i],
        edge_b1[i][None], edge_b2[i][None],
        edge_ln_g[i][None], edge_ln_b[i][None])
    aggs = scatter_agg(edge_feats, dst)
    node_feats = node_mlp(
        node_feats, aggs[0], aggs[1],
        node_W1[i, :D], node_W1[i, D:], node_W2[i],
        node_b1[i][None], node_b2[i][None],
        node_ln_g[i][None], node_ln_b[i][None])

  return (edge_feats, node_feats)


# speedup vs baseline: 3.4813x; 1.6405x over previous
import functools

import jax
import jax.numpy as jnp
from jax import lax
from jax.experimental import pallas as pl
from jax.experimental.pallas import tpu as pltpu
from jax.experimental.pallas import tpu_sc as plsc

_NC = 2
_NS = 16
_NW = _NC * _NS
_LANE = 16

_LN_EPS = 1e-5


def _sc_mesh():
  return plsc.VectorSubcoreMesh(
      core_axis_name="c", subcore_axis_name="s",
      num_cores=_NC, num_subcores=_NS)


_NBUF = 5
_OBUF = _NBUF


def _make_gather_add(E, D, CH):
  ew = E // _NW
  nchunk = ew // CH
  assert nchunk % _NBUF == 0
  nouter = nchunk // _NBUF

  @functools.partial(
      pl.kernel,
      out_type=jax.ShapeDtypeStruct((E, D), jnp.float32),
      mesh=_sc_mesh(),
      scratch_types=(
          [pltpu.VMEM((_NBUF, 1, CH), jnp.int32),
           pltpu.VMEM((_NBUF, 1, CH), jnp.int32),
           pltpu.VMEM((_NBUF, CH, D), jnp.float32),
           pltpu.VMEM((_NBUF, CH, D), jnp.float32),
           pltpu.VMEM((_OBUF, CH, D), jnp.float32)]
          + [pltpu.SemaphoreType.DMA] * (3 * _NBUF + _OBUF)
      ),
  )
  def gather_kernel(p_hbm, q_hbm, src_hbm, dst_hbm, out_hbm,
                    idx_s, idx_d, rows_p, rows_q, obuf, *sems):
    sem_p = sems[:_NBUF]
    sem_q = sems[_NBUF:2 * _NBUF]
    sem_i = sems[2 * _NBUF:3 * _NBUF]
    sem_w = sems[3 * _NBUF:]
    c = lax.axis_index("c")
    s = lax.axis_index("s")
    wid = s * _NC + c
    base = wid * ew

    def issue_idx(j, b):
      pltpu.async_copy(src_hbm.at[wid, j], idx_s.at[b], sem_i[b])
      pltpu.async_copy(dst_hbm.at[wid, j], idx_d.at[b], sem_i[b])

    def wait_idx(j, b):
      pltpu.make_async_copy(src_hbm.at[wid, j], idx_s.at[b], sem_i[b]).wait()
      pltpu.make_async_copy(dst_hbm.at[wid, j], idx_d.at[b], sem_i[b]).wait()

    def issue_gather(b):
      pltpu.async_copy(p_hbm.at[idx_s.at[b, 0]], rows_p.at[b], sem_p[b])
      pltpu.async_copy(q_hbm.at[idx_d.at[b, 0]], rows_q.at[b], sem_q[b])

    for b in range(_NBUF):
      issue_idx(b, b)

    def outer(jj, carry):
      for b in range(_NBUF):
        j = jj * _NBUF + b
        ob = b

        @pl.when(jj == 0)
        def _prologue():
          wait_idx(j, b)
          issue_gather(b)

        pltpu.make_async_copy(
            p_hbm.at[idx_s.at[b, 0]], rows_p.at[b], sem_p[b]).wait()
        pltpu.make_async_copy(
            q_hbm.at[idx_d.at[b, 0]], rows_q.at[b], sem_q[b]).wait()

        @pl.when(j + _NBUF < nchunk)
        def _prefetch_idx():
          issue_idx(j + _NBUF, b)

        @pl.when(j >= _NBUF)
        def _wait_writeout():
          off = pl.multiple_of(base + (j - _NBUF) * CH, 8)
          pltpu.make_async_copy(
              obuf.at[ob], out_hbm.at[pl.ds(off, CH)], sem_w[ob]).wait()

        def add_row(r, carry2):
          for k in range(D // _LANE):
            sl = pl.ds(k * _LANE, _LANE)
            obuf[ob, r, sl] = rows_p[b, r, sl] + rows_q[b, r, sl]
          return carry2

        lax.fori_loop(0, CH, add_row, 0, unroll=2)

        off = pl.multiple_of(base + j * CH, 8)
        pltpu.async_copy(obuf.at[ob], out_hbm.at[pl.ds(off, CH)], sem_w[ob])

        @pl.when(j + _NBUF < nchunk)
        def _next_gather():
          wait_idx(j + _NBUF, b)
          issue_gather(b)
      return carry

    lax.fori_loop(0, nouter, outer, 0)

    for t in range(_NBUF):
      j = nchunk - _NBUF + t
      ob = j % _NBUF
      off = pl.multiple_of(base + j * CH, 8)
      pltpu.make_async_copy(
          obuf.at[ob], out_hbm.at[pl.ds(off, CH)], sem_w[ob]).wait()

  return gather_kernel


def _make_scatter_agg(E, N, D, CH):
  ew = E // _NW
  nchunk = ew // CH
  assert nchunk % _NBUF == 0
  nouter = nchunk // _NBUF
  wsub = 10
  rows_per_sub = N // wsub
  assert wsub * rows_per_sub == N

  @functools.partial(
      pl.kernel,
      out_type=jax.ShapeDtypeStruct((_NC, N, D), jnp.float32),
      mesh=_sc_mesh(),
      scratch_types=(
          [pltpu.VMEM((_NBUF, 1, CH), jnp.int32),
           pltpu.VMEM((_NBUF, CH, D), jnp.float32),
           pltpu.VMEM_SHARED((N, D), jnp.float32)]
          + [pltpu.SemaphoreType.DMA] * (2 * _NBUF)
      ),
  )
  def scatter_kernel(edge_hbm, dst_hbm, zeros_hbm, out_hbm, idx_v, rows_v,
                     agg_sh, *sems):
    sem_r = sems[:_NBUF]
    sem_i = sems[_NBUF:]
    c = lax.axis_index("c")
    s = lax.axis_index("s")
    wid = s * _NC + c
    base = wid * ew

    def issue(j, b):
      off = pl.multiple_of(base + j * CH, 8)
      pltpu.async_copy(edge_hbm.at[pl.ds(off, CH)], rows_v.at[b], sem_r[b])
      pltpu.async_copy(dst_hbm.at[wid, j], idx_v.at[b], sem_i[b])

    for b in range(_NBUF):
      issue(b, b)

    @pl.when(s < wsub)
    def _zero():
      r0 = pl.multiple_of(s * rows_per_sub, 8)
      pltpu.sync_copy(zeros_hbm.at[pl.ds(r0, rows_per_sub)],
                      agg_sh.at[pl.ds(r0, rows_per_sub)])

    plsc.subcore_barrier()

    def outer(jj, carry):
      for b in range(_NBUF):
        j = jj * _NBUF + b
        off = pl.multiple_of(base + j * CH, 8)
        pltpu.make_async_copy(
            edge_hbm.at[pl.ds(off, CH)], rows_v.at[b], sem_r[b]).wait()
        pltpu.make_async_copy(
            dst_hbm.at[wid, j], idx_v.at[b], sem_i[b]).wait()
        pltpu.sync_copy(rows_v.at[b], agg_sh.at[idx_v.at[b, 0]], add=True)

        @pl.when(j + _NBUF < nchunk)
        def _prefetch():
          issue(j + _NBUF, b)
      return carry

    lax.fori_loop(0, nouter, outer, 0)
    plsc.subcore_barrier()

    @pl.when(s < wsub)
    def _writeout():
      r0 = pl.multiple_of(s * rows_per_sub, 8)
      pltpu.sync_copy(agg_sh.at[pl.ds(r0, rows_per_sub)],
                      out_hbm.at[c, pl.ds(r0, rows_per_sub)])

  return scatter_kernel


def _pq_body(x_ref, ws_ref, wd_ref, p_ref, q_ref):
  x = x_ref[...]
  p_ref[...] = jnp.dot(x, ws_ref[...], preferred_element_type=jnp.float32)
  q_ref[...] = jnp.dot(x, wd_ref[...], preferred_element_type=jnp.float32)


def _layer_norm(h, g, b):
  m = jnp.mean(h, axis=-1, keepdims=True)
  xc = h - m
  v = jnp.mean(xc * xc, axis=-1, keepdims=True)
  return xc * lax.rsqrt(v + _LN_EPS) * g + b


def _edge_mlp_body(e_ref, g_ref, w1_ref, w2_ref, b1_ref, b2_ref,
                   lng_ref, lnb_ref, out_ref):
  x = e_ref[...]
  cvt = jnp.dot(x, w1_ref[...], preferred_element_type=jnp.float32)
  cvt = cvt + g_ref[...] + b1_ref[...]
  h = cvt * jax.nn.sigmoid(cvt)
  y = jnp.dot(h, w2_ref[...], preferred_element_type=jnp.float32) + b2_ref[...]
  out_ref[...] = x + _layer_norm(y, lng_ref[...], lnb_ref[...])


def _node_mlp_body(x_ref, a0_ref, a1_ref, w1x_ref, w1a_ref, w2_ref,
                   b1_ref, b2_ref, lng_ref, lnb_ref, out_ref):
  x = x_ref[...]
  a = a0_ref[...] + a1_ref[...]
  cvt = jnp.dot(x, w1x_ref[...], preferred_element_type=jnp.float32)
  cvt = cvt + jnp.dot(a, w1a_ref[...], preferred_element_type=jnp.float32)
  cvt = cvt + b1_ref[...]
  h = cvt * jax.nn.sigmoid(cvt)
  y = jnp.dot(h, w2_ref[...], preferred_element_type=jnp.float32) + b2_ref[...]
  out_ref[...] = x + _layer_norm(y, lng_ref[...], lnb_ref[...])


def _mat_spec(D):
  return pl.BlockSpec((D, D), lambda i: (0, 0))


def _vec_spec(D):
  return pl.BlockSpec((1, D), lambda i: (0, 0))


def kernel(edge_feats, node_feats, edge_index,
           edge_W1, edge_b1, edge_W2, edge_b2, edge_ln_g, edge_ln_b,
           node_W1, node_b1, node_W2, node_b2, node_ln_g, node_ln_b):
  E, D = edge_feats.shape
  N = node_feats.shape[0]
  L = edge_W1.shape[0]
  CH = 40
  nchunk = E // _NW // CH
  src4 = edge_index[0].reshape(_NW, nchunk, 1, CH)
  dst4 = edge_index[1].reshape(_NW, nchunk, 1, CH)
  zeros_nd = jnp.zeros((N, D), jnp.float32)
  EBLK = 2000
  NBLK = 1000

  gather_add = _make_gather_add(E, D, CH)
  scatter_agg = _make_scatter_agg(E, N, D, CH)

  pq_call = pl.pallas_call(
      _pq_body,
      grid=(N // NBLK,),
      in_specs=[pl.BlockSpec((NBLK, D), lambda i: (i, 0)),
                _mat_spec(D), _mat_spec(D)],
      out_specs=[pl.BlockSpec((NBLK, D), lambda i: (i, 0)),
                 pl.BlockSpec((NBLK, D), lambda i: (i, 0))],
      out_shape=[jax.ShapeDtypeStruct((N, D), jnp.float32),
                 jax.ShapeDtypeStruct((N, D), jnp.float32)],
  )

  edge_mlp = pl.pallas_call(
      _edge_mlp_body,
      grid=(E // EBLK,),
      in_specs=[pl.BlockSpec((EBLK, D), lambda i: (i, 0)),
                pl.BlockSpec((EBLK, D), lambda i: (i, 0)),
                _mat_spec(D), _mat_spec(D),
                _vec_spec(D), _vec_spec(D), _vec_spec(D), _vec_spec(D)],
      out_specs=pl.BlockSpec((EBLK, D), lambda i: (i, 0)),
      out_shape=jax.ShapeDtypeStruct((E, D), jnp.float32),
  )

  node_mlp = pl.pallas_call(
      _node_mlp_body,
      grid=(N // NBLK,),
      in_specs=[pl.BlockSpec((NBLK, D), lambda i: (i, 0)),
                pl.BlockSpec((NBLK, D), lambda i: (i, 0)),
                pl.BlockSpec((NBLK, D), lambda i: (i, 0)),
                _mat_spec(D), _mat_spec(D), _mat_spec(D),
                _vec_spec(D), _vec_spec(D), _vec_spec(D), _vec_spec(D)],
      out_specs=pl.BlockSpec((NBLK, D), lambda i: (i, 0)),
      out_shape=jax.ShapeDtypeStruct((N, D), jnp.float32),
  )

  for i in range(L):
    w1e = edge_W1[i, :D]
    w1s = edge_W1[i, D:2 * D]
    w1d = edge_W1[i, 2 * D:]
    p, q = pq_call(node_feats, w1s, w1d)
    g = gather_add(p, q, src4, dst4)
    edge_feats = edge_mlp(
        edge_feats, g, w1e, edge_W2[i],
        edge_b1[i][None], edge_b2[i][None],
        edge_ln_g[i][None], edge_ln_b[i][None])
    aggs = scatter_agg(edge_feats, dst4, zeros_nd)
    node_feats = node_mlp(
        node_feats, aggs[0], aggs[1],
        node_W1[i, :D], node_W1[i, D:], node_W2[i],
        node_b1[i][None], node_b2[i][None],
        node_ln_g[i][None], node_ln_b[i][None])

  return (edge_feats, node_feats)


# speedup vs baseline: 3.5621x; 1.0232x over previous
import functools

import jax
import jax.numpy as jnp
from jax import lax
from jax.experimental import pallas as pl
from jax.experimental.pallas import tpu as pltpu
from jax.experimental.pallas import tpu_sc as plsc

_NC = 2
_NS = 16
_NW = _NC * _NS
_LANE = 16

_LN_EPS = 1e-5


def _sc_mesh():
  return plsc.VectorSubcoreMesh(
      core_axis_name="c", subcore_axis_name="s",
      num_cores=_NC, num_subcores=_NS)


_NBUF = 5
_OBUF = _NBUF


def _make_gather_add(E, D, CH):
  ew = E // _NW
  nchunk = ew // CH
  assert nchunk % _NBUF == 0
  nouter = nchunk // _NBUF

  @functools.partial(
      pl.kernel,
      out_type=jax.ShapeDtypeStruct((E, D), jnp.float32),
      mesh=_sc_mesh(),
      scratch_types=(
          [pltpu.VMEM((2 * _NBUF, 1, CH), jnp.int32),
           pltpu.VMEM((2 * _NBUF, 1, CH), jnp.int32),
           pltpu.VMEM((_NBUF, CH, D), jnp.float32),
           pltpu.VMEM((_NBUF, CH, D), jnp.float32),
           pltpu.VMEM((_OBUF, CH, D), jnp.float32)]
          + [pltpu.SemaphoreType.DMA] * (4 * _NBUF + _OBUF)
      ),
  )
  def gather_kernel(p_hbm, q_hbm, src_hbm, dst_hbm, out_hbm,
                    idx_s, idx_d, rows_p, rows_q, obuf, *sems):
    sem_p = sems[:_NBUF]
    sem_q = sems[_NBUF:2 * _NBUF]
    sem_i = sems[2 * _NBUF:4 * _NBUF]
    sem_w = sems[4 * _NBUF:]
    c = lax.axis_index("c")
    s = lax.axis_index("s")
    wid = s * _NC + c
    base = wid * ew

    def issue_idx(j, u):
      pltpu.async_copy(src_hbm.at[wid, j], idx_s.at[u], sem_i[u])
      pltpu.async_copy(dst_hbm.at[wid, j], idx_d.at[u], sem_i[u])

    def wait_idx(j, u):
      pltpu.make_async_copy(src_hbm.at[wid, j], idx_s.at[u], sem_i[u]).wait()
      pltpu.make_async_copy(dst_hbm.at[wid, j], idx_d.at[u], sem_i[u]).wait()

    def issue_gather(u, b):
      pltpu.async_copy(p_hbm.at[idx_s.at[u, 0]], rows_p.at[b], sem_p[b])
      pltpu.async_copy(q_hbm.at[idx_d.at[u, 0]], rows_q.at[b], sem_q[b])

    for u in range(2 * _NBUF):
      issue_idx(u, u)
    for b in range(_NBUF):
      wait_idx(b, b)
      issue_gather(b, b)

    def half_body(jj, half):
      for b in range(_NBUF):
        j = jj * _NBUF + b
        u = half * _NBUF + b
        u_next = (1 - half) * _NBUF + b
        ob = b

        pltpu.make_async_copy(
            p_hbm.at[idx_s.at[u, 0]], rows_p.at[b], sem_p[b]).wait()
        pltpu.make_async_copy(
            q_hbm.at[idx_d.at[u, 0]], rows_q.at[b], sem_q[b]).wait()

        @pl.when(j + 2 * _NBUF < nchunk)
        def _prefetch_idx():
          issue_idx(j + 2 * _NBUF, u)

        @pl.when(j >= _NBUF)
        def _wait_writeout():
          off = pl.multiple_of(base + (j - _NBUF) * CH, 8)
          pltpu.make_async_copy(
              obuf.at[ob], out_hbm.at[pl.ds(off, CH)], sem_w[ob]).wait()

        def add_row(r, carry2):
          for k in range(D // _LANE):
            sl = pl.ds(k * _LANE, _LANE)
            obuf[ob, r, sl] = rows_p[b, r, sl] + rows_q[b, r, sl]
          return carry2

        lax.fori_loop(0, CH, add_row, 0, unroll=2)

        off = pl.multiple_of(base + j * CH, 8)
        pltpu.async_copy(obuf.at[ob], out_hbm.at[pl.ds(off, CH)], sem_w[ob])

        @pl.when(j + _NBUF < nchunk)
        def _next_gather():
          wait_idx(j + _NBUF, u_next)
          issue_gather(u_next, b)

    def outer(jj, carry):
      @pl.when(jj % 2 == 0)
      def _even():
        half_body(jj, 0)

      @pl.when(jj % 2 == 1)
      def _odd():
        half_body(jj, 1)

      return carry

    lax.fori_loop(0, nouter, outer, 0)

    for t in range(_NBUF):
      j = nchunk - _NBUF + t
      ob = j % _NBUF
      off = pl.multiple_of(base + j * CH, 8)
      pltpu.make_async_copy(
          obuf.at[ob], out_hbm.at[pl.ds(off, CH)], sem_w[ob]).wait()

  return gather_kernel


def _make_scatter_agg(E, N, D, CH):
  ew = E // _NW
  nchunk = ew // CH
  assert nchunk % _NBUF == 0
  nouter = nchunk // _NBUF
  wsub = 10
  rows_per_sub = N // wsub
  assert wsub * rows_per_sub == N

  @functools.partial(
      pl.kernel,
      out_type=jax.ShapeDtypeStruct((_NC, N, D), jnp.float32),
      mesh=_sc_mesh(),
      scratch_types=(
          [pltpu.VMEM((_NBUF, 1, CH), jnp.int32),
           pltpu.VMEM((_NBUF, CH, D), jnp.float32),
           pltpu.VMEM_SHARED((N, D), jnp.float32)]
          + [pltpu.SemaphoreType.DMA] * (2 * _NBUF)
      ),
  )
  def scatter_kernel(edge_hbm, dst_hbm, zeros_hbm, out_hbm, idx_v, rows_v,
                     agg_sh, *sems):
    sem_r = sems[:_NBUF]
    sem_i = sems[_NBUF:]
    c = lax.axis_index("c")
    s = lax.axis_index("s")
    wid = s * _NC + c
    base = wid * ew

    def issue(j, b):
      off = pl.multiple_of(base + j * CH, 8)
      pltpu.async_copy(edge_hbm.at[pl.ds(off, CH)], rows_v.at[b], sem_r[b])
      pltpu.async_copy(dst_hbm.at[wid, j], idx_v.at[b], sem_i[b])

    for b in range(_NBUF):
      issue(b, b)

    @pl.when(s < wsub)
    def _zero():
      r0 = pl.multiple_of(s * rows_per_sub, 8)
      pltpu.sync_copy(zeros_hbm.at[pl.ds(r0, rows_per_sub)],
                      agg_sh.at[pl.ds(r0, rows_per_sub)])

    plsc.subcore_barrier()

    def outer(jj, carry):
      for b in range(_NBUF):
        j = jj * _NBUF + b
        off = pl.multiple_of(base + j * CH, 8)
        pltpu.make_async_copy(
            edge_hbm.at[pl.ds(off, CH)], rows_v.at[b], sem_r[b]).wait()
        pltpu.make_async_copy(
            dst_hbm.at[wid, j], idx_v.at[b], sem_i[b]).wait()
        pltpu.sync_copy(rows_v.at[b], agg_sh.at[idx_v.at[b, 0]], add=True)

        @pl.when(j + _NBUF < nchunk)
        def _prefetch():
          issue(j + _NBUF, b)
      return carry

    lax.fori_loop(0, nouter, outer, 0)
    plsc.subcore_barrier()

    @pl.when(s < wsub)
    def _writeout():
      r0 = pl.multiple_of(s * rows_per_sub, 8)
      pltpu.sync_copy(agg_sh.at[pl.ds(r0, rows_per_sub)],
                      out_hbm.at[c, pl.ds(r0, rows_per_sub)])

  return scatter_kernel


def _pq_body(x_ref, ws_ref, wd_ref, p_ref, q_ref):
  x = x_ref[...]
  p_ref[...] = jnp.dot(x, ws_ref[...], preferred_element_type=jnp.float32)
  q_ref[...] = jnp.dot(x, wd_ref[...], preferred_element_type=jnp.float32)


def _layer_norm(h, g, b):
  m = jnp.mean(h, axis=-1, keepdims=True)
  xc = h - m
  v = jnp.mean(xc * xc, axis=-1, keepdims=True)
  return xc * lax.rsqrt(v + _LN_EPS) * g + b


def _edge_mlp_body(e_ref, g_ref, w1_ref, w2_ref, b1_ref, b2_ref,
                   lng_ref, lnb_ref, out_ref):
  x = e_ref[...]
  cvt = jnp.dot(x, w1_ref[...], preferred_element_type=jnp.float32)
  cvt = cvt + g_ref[...] + b1_ref[...]
  h = cvt * jax.nn.sigmoid(cvt)
  y = jnp.dot(h, w2_ref[...], preferred_element_type=jnp.float32) + b2_ref[...]
  out_ref[...] = x + _layer_norm(y, lng_ref[...], lnb_ref[...])


def _node_mlp_body(x_ref, a0_ref, a1_ref, w1x_ref, w1a_ref, w2_ref,
                   b1_ref, b2_ref, lng_ref, lnb_ref, out_ref):
  x = x_ref[...]
  a = a0_ref[...] + a1_ref[...]
  cvt = jnp.dot(x, w1x_ref[...], preferred_element_type=jnp.float32)
  cvt = cvt + jnp.dot(a, w1a_ref[...], preferred_element_type=jnp.float32)
  cvt = cvt + b1_ref[...]
  h = cvt * jax.nn.sigmoid(cvt)
  y = jnp.dot(h, w2_ref[...], preferred_element_type=jnp.float32) + b2_ref[...]
  out_ref[...] = x + _layer_norm(y, lng_ref[...], lnb_ref[...])


def _mat_spec(D):
  return pl.BlockSpec((D, D), lambda i: (0, 0))


def _vec_spec(D):
  return pl.BlockSpec((1, D), lambda i: (0, 0))


def kernel(edge_feats, node_feats, edge_index,
           edge_W1, edge_b1, edge_W2, edge_b2, edge_ln_g, edge_ln_b,
           node_W1, node_b1, node_W2, node_b2, node_ln_g, node_ln_b):
  E, D = edge_feats.shape
  N = node_feats.shape[0]
  L = edge_W1.shape[0]
  CH = 40
  nchunk = E // _NW // CH
  src4 = edge_index[0].reshape(_NW, nchunk, 1, CH)
  dst4 = edge_index[1].reshape(_NW, nchunk, 1, CH)
  zeros_nd = jnp.zeros((N, D), jnp.float32)
  EBLK = 2000
  NBLK = 1000

  gather_add = _make_gather_add(E, D, CH)
  scatter_agg = _make_scatter_agg(E, N, D, CH)

  pq_call = pl.pallas_call(
      _pq_body,
      grid=(N // NBLK,),
      in_specs=[pl.BlockSpec((NBLK, D), lambda i: (i, 0)),
                _mat_spec(D), _mat_spec(D)],
      out_specs=[pl.BlockSpec((NBLK, D), lambda i: (i, 0)),
                 pl.BlockSpec((NBLK, D), lambda i: (i, 0))],
      out_shape=[jax.ShapeDtypeStruct((N, D), jnp.float32),
                 jax.ShapeDtypeStruct((N, D), jnp.float32)],
  )

  edge_mlp = pl.pallas_call(
      _edge_mlp_body,
      grid=(E // EBLK,),
      in_specs=[pl.BlockSpec((EBLK, D), lambda i: (i, 0)),
                pl.BlockSpec((EBLK, D), lambda i: (i, 0)),
                _mat_spec(D), _mat_spec(D),
                _vec_spec(D), _vec_spec(D), _vec_spec(D), _vec_spec(D)],
      out_specs=pl.BlockSpec((EBLK, D), lambda i: (i, 0)),
      out_shape=jax.ShapeDtypeStruct((E, D), jnp.float32),
  )

  node_mlp = pl.pallas_call(
      _node_mlp_body,
      grid=(N // NBLK,),
      in_specs=[pl.BlockSpec((NBLK, D), lambda i: (i, 0)),
                pl.BlockSpec((NBLK, D), lambda i: (i, 0)),
                pl.BlockSpec((NBLK, D), lambda i: (i, 0)),
                _mat_spec(D), _mat_spec(D), _mat_spec(D),
                _vec_spec(D), _vec_spec(D), _vec_spec(D), _vec_spec(D)],
      out_specs=pl.BlockSpec((NBLK, D), lambda i: (i, 0)),
      out_shape=jax.ShapeDtypeStruct((N, D), jnp.float32),
  )

  for i in range(L):
    w1e = edge_W1[i, :D]
    w1s = edge_W1[i, D:2 * D]
    w1d = edge_W1[i, 2 * D:]
    p, q = pq_call(node_feats, w1s, w1d)
    g = gather_add(p, q, src4, dst4)
    edge_feats = edge_mlp(
        edge_feats, g, w1e, edge_W2[i],
        edge_b1[i][None], edge_b2[i][None],
        edge_ln_g[i][None], edge_ln_b[i][None])
    aggs = scatter_agg(edge_feats, dst4, zeros_nd)
    node_feats = node_mlp(
        node_feats, aggs[0], aggs[1],
        node_W1[i, :D], node_W1[i, D:], node_W2[i],
        node_b1[i][None], node_b2[i][None],
        node_ln_g[i][None], node_ln_b[i][None])

  return (edge_feats, node_feats)


# speedup vs baseline: 4.1611x; 1.1682x over previous
import functools

import jax
import jax.numpy as jnp
from jax import lax
from jax.experimental import pallas as pl
from jax.experimental.pallas import tpu as pltpu
from jax.experimental.pallas import tpu_sc as plsc

_NC = 2
_NS = 16
_NW = _NC * _NS
_LANE = 16

_LN_EPS = 1e-5


def _sc_mesh():
  return plsc.VectorSubcoreMesh(
      core_axis_name="c", subcore_axis_name="s",
      num_cores=_NC, num_subcores=_NS)


_NBUF = 5
_OBUF = _NBUF


def _make_gather_add(E, D, CH):
  ew = E // _NW
  nchunk = ew // CH
  assert nchunk % _NBUF == 0
  nouter = nchunk // _NBUF

  @functools.partial(
      pl.kernel,
      out_type=jax.ShapeDtypeStruct((E, D), jnp.float32),
      mesh=_sc_mesh(),
      scratch_types=(
          [pltpu.VMEM((2 * _NBUF, 1, CH), jnp.int32),
           pltpu.VMEM((2 * _NBUF, 1, CH), jnp.int32),
           pltpu.VMEM((_NBUF, CH, D), jnp.float32),
           pltpu.VMEM((_NBUF, CH, D), jnp.float32),
           pltpu.VMEM((_OBUF, CH, D), jnp.float32)]
          + [pltpu.SemaphoreType.DMA] * (4 * _NBUF + _OBUF)
      ),
  )
  def gather_kernel(p_hbm, q_hbm, src_hbm, dst_hbm, out_hbm,
                    idx_s, idx_d, rows_p, rows_q, obuf, *sems):
    sem_p = sems[:_NBUF]
    sem_q = sems[_NBUF:2 * _NBUF]
    sem_i = sems[2 * _NBUF:4 * _NBUF]
    sem_w = sems[4 * _NBUF:]
    c = lax.axis_index("c")
    s = lax.axis_index("s")
    wid = s * _NC + c
    base = wid * ew

    def issue_idx(j, u):
      pltpu.async_copy(src_hbm.at[wid, j], idx_s.at[u], sem_i[u])
      pltpu.async_copy(dst_hbm.at[wid, j], idx_d.at[u], sem_i[u])

    def wait_idx(j, u):
      pltpu.make_async_copy(src_hbm.at[wid, j], idx_s.at[u], sem_i[u]).wait()
      pltpu.make_async_copy(dst_hbm.at[wid, j], idx_d.at[u], sem_i[u]).wait()

    def issue_gather(u, b):
      pltpu.async_copy(p_hbm.at[idx_s.at[u, 0]], rows_p.at[b], sem_p[b])
      pltpu.async_copy(q_hbm.at[idx_d.at[u, 0]], rows_q.at[b], sem_q[b])

    for u in range(2 * _NBUF):
      issue_idx(u, u)
    for b in range(_NBUF):
      wait_idx(b, b)
      issue_gather(b, b)

    def half_body(jj, half):
      for b in range(_NBUF):
        j = jj * _NBUF + b
        u = half * _NBUF + b
        u_next = (1 - half) * _NBUF + b
        ob = b

        pltpu.make_async_copy(
            p_hbm.at[idx_s.at[u, 0]], rows_p.at[b], sem_p[b]).wait()
        pltpu.make_async_copy(
            q_hbm.at[idx_d.at[u, 0]], rows_q.at[b], sem_q[b]).wait()

        @pl.when(j + 2 * _NBUF < nchunk)
        def _prefetch_idx():
          issue_idx(j + 2 * _NBUF, u)

        @pl.when(j >= _NBUF)
        def _wait_writeout():
          off = pl.multiple_of(base + (j - _NBUF) * CH, 8)
          pltpu.make_async_copy(
              obuf.at[ob], out_hbm.at[pl.ds(off, CH)], sem_w[ob]).wait()

        def add_row(r, carry2):
          for k in range(D // _LANE):
            sl = pl.ds(k * _LANE, _LANE)
            obuf[ob, r, sl] = rows_p[b, r, sl] + rows_q[b, r, sl]
          return carry2

        lax.fori_loop(0, CH, add_row, 0, unroll=2)

        off = pl.multiple_of(base + j * CH, 8)
        pltpu.async_copy(obuf.at[ob], out_hbm.at[pl.ds(off, CH)], sem_w[ob])

        @pl.when(j + _NBUF < nchunk)
        def _next_gather():
          wait_idx(j + _NBUF, u_next)
          issue_gather(u_next, b)

    def outer(jj, carry):
      @pl.when(jj % 2 == 0)
      def _even():
        half_body(jj, 0)

      @pl.when(jj % 2 == 1)
      def _odd():
        half_body(jj, 1)

      return carry

    lax.fori_loop(0, nouter, outer, 0)

    for t in range(_NBUF):
      j = nchunk - _NBUF + t
      ob = j % _NBUF
      off = pl.multiple_of(base + j * CH, 8)
      pltpu.make_async_copy(
          obuf.at[ob], out_hbm.at[pl.ds(off, CH)], sem_w[ob]).wait()

  return gather_kernel


def _make_scatter_agg(E, N, D, CH):
  ew = E // _NW
  nchunk = ew // CH
  assert nchunk % _NBUF == 0
  nouter = nchunk // _NBUF
  wsub = 10
  rows_per_sub = N // wsub
  assert wsub * rows_per_sub == N

  @functools.partial(
      pl.kernel,
      out_type=jax.ShapeDtypeStruct((_NC, N, D), jnp.float32),
      mesh=_sc_mesh(),
      scratch_types=(
          [pltpu.VMEM((_NBUF, 1, CH), jnp.int32),
           pltpu.VMEM((_NBUF, CH, D), jnp.float32),
           pltpu.VMEM_SHARED((N, D), jnp.float32)]
          + [pltpu.SemaphoreType.DMA] * (2 * _NBUF)
      ),
  )
  def scatter_kernel(edge_hbm, dst_hbm, zeros_hbm, out_hbm, idx_v, rows_v,
                     agg_sh, *sems):
    sem_r = sems[:_NBUF]
    sem_i = sems[_NBUF:]
    c = lax.axis_index("c")
    s = lax.axis_index("s")
    wid = s * _NC + c
    base = wid * ew

    def issue(j, b):
      off = pl.multiple_of(base + j * CH, 8)
      pltpu.async_copy(edge_hbm.at[pl.ds(off, CH)], rows_v.at[b], sem_r[b])
      pltpu.async_copy(dst_hbm.at[wid, j], idx_v.at[b], sem_i[b])

    for b in range(_NBUF):
      issue(b, b)

    @pl.when(s < wsub)
    def _zero():
      r0 = pl.multiple_of(s * rows_per_sub, 8)
      pltpu.sync_copy(zeros_hbm.at[pl.ds(r0, rows_per_sub)],
                      agg_sh.at[pl.ds(r0, rows_per_sub)])

    plsc.subcore_barrier()

    def outer(jj, carry):
      for b in range(_NBUF):
        j = jj * _NBUF + b
        off = pl.multiple_of(base + j * CH, 8)
        pltpu.make_async_copy(
            edge_hbm.at[pl.ds(off, CH)], rows_v.at[b], sem_r[b]).wait()
        pltpu.make_async_copy(
            dst_hbm.at[wid, j], idx_v.at[b], sem_i[b]).wait()
        pltpu.sync_copy(rows_v.at[b], agg_sh.at[idx_v.at[b, 0]], add=True)

        @pl.when(j + _NBUF < nchunk)
        def _prefetch():
          issue(j + _NBUF, b)
      return carry

    lax.fori_loop(0, nouter, outer, 0)
    plsc.subcore_barrier()

    @pl.when(s < wsub)
    def _writeout():
      r0 = pl.multiple_of(s * rows_per_sub, 8)
      pltpu.sync_copy(agg_sh.at[pl.ds(r0, rows_per_sub)],
                      out_hbm.at[c, pl.ds(r0, rows_per_sub)])

  return scatter_kernel


def _pq_body(x_ref, ws_ref, wd_ref, p_ref, q_ref):
  x = x_ref[...]
  p_ref[...] = jnp.dot(x, ws_ref[...], preferred_element_type=jnp.float32)
  q_ref[...] = jnp.dot(x, wd_ref[...], preferred_element_type=jnp.float32)


def _layer_norm(h, g, b):
  m = jnp.mean(h, axis=-1, keepdims=True)
  xc = h - m
  v = jnp.mean(xc * xc, axis=-1, keepdims=True)
  return xc * lax.rsqrt(v + _LN_EPS) * g + b


def _edge_mlp_body(e_ref, g_ref, w1_ref, w2_ref, b1_ref, b2_ref,
                   lng_ref, lnb_ref, out_ref):
  x = e_ref[...]
  cvt = jnp.dot(x, w1_ref[...], preferred_element_type=jnp.float32)
  cvt = cvt + g_ref[...] + b1_ref[...]
  h = cvt * jax.nn.sigmoid(cvt)
  y = jnp.dot(h, w2_ref[...], preferred_element_type=jnp.float32) + b2_ref[...]
  out_ref[...] = x + _layer_norm(y, lng_ref[...], lnb_ref[...])


def _node_mlp_body(x_ref, a0_ref, a1_ref, a2_ref, a3_ref,
                   w1x_ref, w1a_ref, w2_ref,
                   b1_ref, b2_ref, lng_ref, lnb_ref, out_ref):
  x = x_ref[...]
  a = (a0_ref[...] + a1_ref[...]) + (a2_ref[...] + a3_ref[...])
  cvt = jnp.dot(x, w1x_ref[...], preferred_element_type=jnp.float32)
  cvt = cvt + jnp.dot(a, w1a_ref[...], preferred_element_type=jnp.float32)
  cvt = cvt + b1_ref[...]
  h = cvt * jax.nn.sigmoid(cvt)
  y = jnp.dot(h, w2_ref[...], preferred_element_type=jnp.float32) + b2_ref[...]
  out_ref[...] = x + _layer_norm(y, lng_ref[...], lnb_ref[...])


def _mat_spec(D):
  return pl.BlockSpec((D, D), lambda i: (0, 0))


def _vec_spec(D):
  return pl.BlockSpec((1, D), lambda i: (0, 0))


def kernel(edge_feats, node_feats, edge_index,
           edge_W1, edge_b1, edge_W2, edge_b2, edge_ln_g, edge_ln_b,
           node_W1, node_b1, node_W2, node_b2, node_ln_g, node_ln_b):
  E, D = edge_feats.shape
  N = node_feats.shape[0]
  L = edge_W1.shape[0]
  CH = 40
  NH = 2
  Eh = E // NH
  nchunk = Eh // _NW // CH
  src4 = [edge_index[0, h * Eh:(h + 1) * Eh].reshape(_NW, nchunk, 1, CH)
          for h in range(NH)]
  dst4 = [edge_index[1, h * Eh:(h + 1) * Eh].reshape(_NW, nchunk, 1, CH)
          for h in range(NH)]
  e_half = [edge_feats[h * Eh:(h + 1) * Eh] for h in range(NH)]
  zeros_nd = jnp.zeros((N, D), jnp.float32)
  EBLK = 2000
  NBLK = 1000

  gather_add = _make_gather_add(Eh, D, CH)
  scatter_agg = _make_scatter_agg(Eh, N, D, CH)

  pq_call = pl.pallas_call(
      _pq_body,
      grid=(N // NBLK,),
      in_specs=[pl.BlockSpec((NBLK, D), lambda i: (i, 0)),
                _mat_spec(D), _mat_spec(D)],
      out_specs=[pl.BlockSpec((NBLK, D), lambda i: (i, 0)),
                 pl.BlockSpec((NBLK, D), lambda i: (i, 0))],
      out_shape=[jax.ShapeDtypeStruct((N, D), jnp.float32),
                 jax.ShapeDtypeStruct((N, D), jnp.float32)],
  )

  edge_mlp = pl.pallas_call(
      _edge_mlp_body,
      grid=(Eh // EBLK,),
      in_specs=[pl.BlockSpec((EBLK, D), lambda i: (i, 0)),
                pl.BlockSpec((EBLK, D), lambda i: (i, 0)),
                _mat_spec(D), _mat_spec(D),
                _vec_spec(D), _vec_spec(D), _vec_spec(D), _vec_spec(D)],
      out_specs=pl.BlockSpec((EBLK, D), lambda i: (i, 0)),
      out_shape=jax.ShapeDtypeStruct((Eh, D), jnp.float32),
  )

  node_mlp = pl.pallas_call(
      _node_mlp_body,
      grid=(N // NBLK,),
      in_specs=[pl.BlockSpec((NBLK, D), lambda i: (i, 0)),
                pl.BlockSpec((NBLK, D), lambda i: (i, 0)),
                pl.BlockSpec((NBLK, D), lambda i: (i, 0)),
                pl.BlockSpec((NBLK, D), lambda i: (i, 0)),
                pl.BlockSpec((NBLK, D), lambda i: (i, 0)),
                _mat_spec(D), _mat_spec(D), _mat_spec(D),
                _vec_spec(D), _vec_spec(D), _vec_spec(D), _vec_spec(D)],
      out_specs=pl.BlockSpec((NBLK, D), lambda i: (i, 0)),
      out_shape=jax.ShapeDtypeStruct((N, D), jnp.float32),
  )

  for i in range(L):
    w1e = edge_W1[i, :D]
    w1s = edge_W1[i, D:2 * D]
    w1d = edge_W1[i, 2 * D:]
    p, q = pq_call(node_feats, w1s, w1d)
    g = [gather_add(p, q, src4[h], dst4[h]) for h in range(NH)]
    e_half = [
        edge_mlp(e_half[h], g[h], w1e, edge_W2[i],
                 edge_b1[i][None], edge_b2[i][None],
                 edge_ln_g[i][None], edge_ln_b[i][None])
        for h in range(NH)]
    aggs = [scatter_agg(e_half[h], dst4[h], zeros_nd) for h in range(NH)]
    node_feats = node_mlp(
        node_feats, aggs[0][0], aggs[0][1], aggs[1][0], aggs[1][1],
        node_W1[i, :D], node_W1[i, D:], node_W2[i],
        node_b1[i][None], node_b2[i][None],
        node_ln_g[i][None], node_ln_b[i][None])

  return (jnp.concatenate(e_half, axis=0), node_feats)
